# baseline (speedup 1.0000x reference)
import functools

import jax
import jax.numpy as jnp
from jax.experimental import pallas as pl
from jax.experimental.pallas import tpu as pltpu


def _pool(x, h, w):
    parts = [x[i, j] for i in range(h) for j in range(w)]
    while len(parts) > 1:
        nxt = [a + b for a, b in zip(parts[::2], parts[1::2])]
        if len(parts) % 2:
            nxt.append(parts[-1])
        parts = nxt
    return parts[0] * (1.0 / (h * w))


def _body(xa_ref, xb_ref, wlt_ref, bl_ref, wdt_ref, bd_ref,
          logits_ref, probs_ref, deltas_ref, *, h, w):
    acc_a = _pool(xa_ref[...], h, w)
    acc_b = _pool(xb_ref[...], h, w)
    acc_t = jnp.concatenate(
        [jnp.transpose(acc_a), jnp.transpose(acc_b)], axis=1
    )
    logits_t = (
        jnp.dot(wlt_ref[...], acc_t, preferred_element_type=jnp.float32)
        + jnp.transpose(bl_ref[...])
    )
    logits_ref[...] = logits_t
    m = jnp.max(logits_t, axis=0, keepdims=True)
    e = jnp.exp(logits_t - m)
    probs_ref[...] = e / jnp.sum(e, axis=0, keepdims=True)
    deltas_ref[...] = (
        jnp.dot(wdt_ref[...], acc_t, preferred_element_type=jnp.float32)
        + jnp.transpose(bd_ref[...])
    )


def kernel(pooled_rois, W_logits, b_logits, W_delta, b_delta):
    n, h, w, ch = pooled_rois.shape
    ncls = W_logits.shape[1]
    nd = W_delta.shape[1]

    blk = 256
    while n % (2 * blk):
        blk //= 2
    grid = (n // (2 * blk),)

    xt = jnp.transpose(pooled_rois, (1, 2, 0, 3))
    wlt = jnp.transpose(W_logits)
    wdt = jnp.transpose(W_delta)
    bl = b_logits.reshape(1, ncls)
    bd = b_delta.reshape(1, nd)

    body = functools.partial(_body, h=h, w=w)
    logits_t, probs_t, deltas_t = pl.pallas_call(
        body,
        grid=grid,
        in_specs=[
            pl.BlockSpec((h, w, blk, ch), lambda i: (0, 0, 2 * i, 0)),
            pl.BlockSpec((h, w, blk, ch), lambda i: (0, 0, 2 * i + 1, 0)),
            pl.BlockSpec((ncls, ch), lambda i: (0, 0)),
            pl.BlockSpec((1, ncls), lambda i: (0, 0)),
            pl.BlockSpec((nd, ch), lambda i: (0, 0)),
            pl.BlockSpec((1, nd), lambda i: (0, 0)),
        ],
        out_specs=[
            pl.BlockSpec((ncls, 2 * blk), lambda i: (0, i)),
            pl.BlockSpec((ncls, 2 * blk), lambda i: (0, i)),
            pl.BlockSpec((nd, 2 * blk), lambda i: (0, i)),
        ],
        out_shape=[
            jax.ShapeDtypeStruct((ncls, n), jnp.float32),
            jax.ShapeDtypeStruct((ncls, n), jnp.float32),
            jax.ShapeDtypeStruct((nd, n), jnp.float32),
        ],
        compiler_params=pltpu.CompilerParams(
            dimension_semantics=("parallel",),
        ),
    )(xt, xt, wlt, bl, wdt, bd)
    return (
        jnp.transpose(logits_t),
        jnp.transpose(probs_t),
        jnp.transpose(deltas_t),
    )


# speedup vs baseline: 1.1568x; 1.1568x over previous
import functools

import jax
import jax.numpy as jnp
from jax.experimental import pallas as pl
from jax.experimental.pallas import tpu as pltpu


def _body(x_ref, wlt_ref, bl_ref, wdt_ref, bd_ref,
          logits_ref, probs_ref, deltas_ref, *, h, w):
    x = x_ref[...]
    parts = [x[i, j] for i in range(h) for j in range(w)]
    while len(parts) > 1:
        nxt = [a + b for a, b in zip(parts[::2], parts[1::2])]
        if len(parts) % 2:
            nxt.append(parts[-1])
        parts = nxt
    acc_t = jnp.transpose(parts[0] * (1.0 / (h * w)))
    logits_t = (
        jnp.dot(wlt_ref[...], acc_t, preferred_element_type=jnp.float32)
        + jnp.transpose(bl_ref[...])
    )
    logits_ref[...] = logits_t
    m = jnp.max(logits_t, axis=0, keepdims=True)
    e = jnp.exp(logits_t - m)
    probs_ref[...] = e / jnp.sum(e, axis=0, keepdims=True)
    deltas_ref[...] = (
        jnp.dot(wdt_ref[...], acc_t, preferred_element_type=jnp.float32)
        + jnp.transpose(bd_ref[...])
    )


def kernel(pooled_rois, W_logits, b_logits, W_delta, b_delta):
    n, h, w, ch = pooled_rois.shape
    ncls = W_logits.shape[1]
    nd = W_delta.shape[1]

    blk = 256
    while n % blk:
        blk //= 2
    grid = (n // blk,)

    xt = jnp.transpose(pooled_rois, (1, 2, 0, 3))
    wlt = jnp.transpose(W_logits)
    wdt = jnp.transpose(W_delta)
    bl = b_logits.reshape(1, ncls)
    bd = b_delta.reshape(1, nd)

    body = functools.partial(_body, h=h, w=w)
    logits_t, probs_t, deltas_t = pl.pallas_call(
        body,
        grid=grid,
        in_specs=[
            pl.BlockSpec((h, w, blk, ch), lambda i: (0, 0, i, 0)),
            pl.BlockSpec((ncls, ch), lambda i: (0, 0)),
            pl.BlockSpec((1, ncls), lambda i: (0, 0)),
            pl.BlockSpec((nd, ch), lambda i: (0, 0)),
            pl.BlockSpec((1, nd), lambda i: (0, 0)),
        ],
        out_specs=[
            pl.BlockSpec((ncls, blk), lambda i: (0, i)),
            pl.BlockSpec((ncls, blk), lambda i: (0, i)),
            pl.BlockSpec((nd, blk), lambda i: (0, i)),
        ],
        out_shape=[
            jax.ShapeDtypeStruct((ncls, n), jnp.float32),
            jax.ShapeDtypeStruct((ncls, n), jnp.float32),
            jax.ShapeDtypeStruct((nd, n), jnp.float32),
        ],
        compiler_params=pltpu.CompilerParams(
            dimension_semantics=("parallel",),
        ),
    )(xt, wlt, bl, wdt, bd)
    return (
        jnp.transpose(logits_t),
        jnp.transpose(probs_t),
        jnp.transpose(deltas_t),
    )
